# baseline (speedup 1.0000x reference)
import functools

import jax
import jax.numpy as jnp
from jax import lax
from jax.experimental import pallas as pl
from jax.experimental.pallas import tpu as pltpu
from jax.experimental.pallas import tpu_sc as plsc

_D = 64
_SCALE = 8.0
_NC = 2
_NS = 16
_NW = _NC * _NS
_IR = 128
_STREAMS = 2
_CHUNK = _IR * _STREAMS
_NBUF = 4
_LEAD = 2
_GRAN = _NW * _CHUNK * _NBUF


def _sc_gather_scale(table, idx3):
    n_idx_rows = idx3.shape[1]
    rows_per_w = n_idx_rows * _IR
    n_chunks = rows_per_w // _CHUNK
    n_groups = n_chunks // _NBUF
    b_total = _NW * rows_per_w

    mesh = plsc.VectorSubcoreMesh(core_axis_name="c", subcore_axis_name="s")

    @functools.partial(
        pl.kernel,
        out_type=jax.ShapeDtypeStruct((b_total, _D), jnp.float32),
        mesh=mesh,
        scratch_types=[
            pltpu.VMEM((n_idx_rows, _IR), jnp.int32),
            *[pltpu.VMEM((_CHUNK, _D), jnp.float32) for _ in range(_NBUF)],
            *[pltpu.SemaphoreType.DMA for _ in range(2 * _NBUF)],
        ],
        compiler_params=pltpu.CompilerParams(use_tc_tiling_on_sc=False),
    )
    def k(table_hbm, idx_hbm, out_hbm, idx_v,
          b0, b1, b2, b3, g0, g1, g2, g3, o0, o1, o2, o3):
        bufs = (b0, b1, b2, b3)
        gsems = (g0, g1, g2, g3)
        osems = (o0, o1, o2, o3)
        wid = lax.axis_index("s") * _NC + lax.axis_index("c")
        base = wid * rows_per_w

        pltpu.sync_copy(idx_hbm.at[wid], idx_v)

        def gather_desc(j, s, q):
            return pltpu.make_async_copy(
                table_hbm.at[idx_v.at[_STREAMS * j + q]],
                bufs[s].at[pl.ds(q * _IR, _IR)],
                gsems[s])

        def store_desc(j, s):
            return pltpu.make_async_copy(
                bufs[s],
                out_hbm.at[pl.ds(base + j * _CHUNK, _CHUNK)],
                osems[s])

        def start_gather(j, s):
            for q in range(_STREAMS):
                gather_desc(j, s, q).start()

        def wait_gather(j, s):
            for q in range(_STREAMS):
                gather_desc(j, s, q).wait()

        for j0 in range(_LEAD):
            start_gather(j0, j0)

        def group(gi, carry):
            for s in range(_NBUF):
                j = gi * _NBUF + s
                ns = (s + _LEAD) % _NBUF

                @pl.when(j + _LEAD < n_chunks)
                def _():
                    @pl.when(j >= _NBUF - _LEAD)
                    def _():
                        store_desc(j - (_NBUF - _LEAD), ns).wait()
                    start_gather(j + _LEAD, ns)

                wait_gather(j, s)

                def scale_rows(i, c):
                    for r in range(8):
                        for q in range(_D // 16):
                            sl = (8 * i + r, pl.ds(q * 16, 16))
                            bufs[s][sl] = bufs[s][sl] * _SCALE
                    return c

                lax.fori_loop(0, _CHUNK // 8, scale_rows, 0)
                store_desc(j, s).start()
            return carry

        lax.fori_loop(0, n_groups, group, 0)

        for m in range(n_chunks - _NBUF, n_chunks):
            store_desc(m, m % _NBUF).wait()

    return k(table, idx3)


def kernel(x, lut_weight):
    n = x.size
    flat = x.reshape(-1).astype(jnp.int32)
    pad = (-n) % _GRAN
    if pad:
        flat = jnp.concatenate([flat, jnp.zeros((pad,), jnp.int32)])
    idx3 = flat.reshape(_NW, -1, _IR)
    out = _sc_gather_scale(lut_weight, idx3)
    if pad:
        out = out[:n]
    return out.reshape(*x.shape, _D)
